# Initial kernel scaffold; baseline (speedup 1.0000x reference)
import jax
import jax.numpy as jnp
from jax.experimental import pallas as pl


def kernel(h_t, W_g, b_g, W1, b1, W2, b2):
    raise NotImplementedError("write your pallas kernel here")


# speedup vs baseline: 6.7164x; 6.7164x over previous
import jax
import jax.numpy as jnp
from jax.experimental import pallas as pl


def _router_kernel(h_ref, wg_ref, bg_ref,
                   logits_ref, probs_ref, gate_ref,
                   zl_ref, lb_ref, load_ref, counts_ref):
    T, E = logits_ref.shape
    K = 2
    h = h_ref[...]
    wg = wg_ref[...]
    logits = jnp.dot(h, wg, preferred_element_type=jnp.float32) + bg_ref[...]
    logits_ref[...] = logits

    m = jnp.max(logits, axis=1, keepdims=True)
    el = jnp.exp(logits - m)
    se = jnp.sum(el, axis=1, keepdims=True)
    probs = el / se
    probs_ref[...] = probs

    idx = jax.lax.broadcasted_iota(jnp.int32, (T, E), 1)
    m1 = jnp.max(probs, axis=1, keepdims=True)
    a1 = jnp.min(jnp.where(probs == m1, idx, E), axis=1, keepdims=True)
    p2 = jnp.where(idx == a1, -jnp.inf, probs)
    m2 = jnp.max(p2, axis=1, keepdims=True)
    a2 = jnp.min(jnp.where(p2 == m2, idx, E), axis=1, keepdims=True)
    s = m1 + m2
    w1 = m1 / s
    w2 = m2 / s
    oh1 = (idx == a1).astype(jnp.float32)
    oh2 = (idx == a2).astype(jnp.float32)
    gate_ref[...] = w1 * oh1 + w2 * oh2

    counts = jnp.sum(oh1 + oh2, axis=0, keepdims=True)
    counts_ref[...] = counts
    load = counts / (T * K)
    load_ref[...] = load

    z = m + jnp.log(se)
    zl_ref[...] = jnp.sum(z * z, axis=0, keepdims=True) / T

    mean_probs = jnp.sum(probs, axis=0, keepdims=True) / T
    lb_ref[...] = jnp.sum(load * mean_probs, axis=1, keepdims=True) * E


def _ffn_kernel(gate_ref, h_ref, w1_ref, b1_ref, w2_ref, b2_ref, out_ref):
    e = pl.program_id(0)
    f = pl.program_id(1)

    @pl.when((e == 0) & (f == 0))
    def _():
        out_ref[...] = jnp.zeros_like(out_ref)

    T, E = gate_ref.shape
    idx = jax.lax.broadcasted_iota(jnp.int32, (T, E), 1)
    gcol = jnp.sum(jnp.where(idx == e, gate_ref[...], 0.0), axis=1,
                   keepdims=True)

    h = h_ref[...]
    mid = jnp.dot(h.astype(jnp.bfloat16), w1_ref[0].astype(jnp.bfloat16),
                  preferred_element_type=jnp.float32) + b1_ref[0]
    mid = jax.nn.gelu(mid) * gcol
    contrib = jnp.dot(mid.astype(jnp.bfloat16), w2_ref[0].astype(jnp.bfloat16),
                      preferred_element_type=jnp.float32)

    @pl.when(f == 0)
    def _():
        out_ref[...] += gcol * b2_ref[0]

    out_ref[...] += contrib


def kernel(h_t, W_g, b_g, W1, b1, W2, b2):
    B, S, D = h_t.shape
    E = W_g.shape[1]
    F = W1.shape[2]
    T = B * S
    FFB = 1024
    NFF = F // FFB

    h = h_t.reshape(T, D)
    logits, probs, gate, zl, lb, load, counts = pl.pallas_call(
        _router_kernel,
        out_shape=(
            jax.ShapeDtypeStruct((T, E), jnp.float32),
            jax.ShapeDtypeStruct((T, E), jnp.float32),
            jax.ShapeDtypeStruct((T, E), jnp.float32),
            jax.ShapeDtypeStruct((1, 1), jnp.float32),
            jax.ShapeDtypeStruct((1, 1), jnp.float32),
            jax.ShapeDtypeStruct((1, E), jnp.float32),
            jax.ShapeDtypeStruct((1, E), jnp.float32),
        ),
    )(h, W_g, b_g.reshape(1, E))

    out = pl.pallas_call(
        _ffn_kernel,
        grid=(E, NFF),
        in_specs=[
            pl.BlockSpec((T, E), lambda e, f: (0, 0)),
            pl.BlockSpec((T, D), lambda e, f: (0, 0)),
            pl.BlockSpec((1, D, FFB), lambda e, f: (e, 0, f)),
            pl.BlockSpec((1, 1, FFB), lambda e, f: (e, 0, f)),
            pl.BlockSpec((1, FFB, D), lambda e, f: (e, f, 0)),
            pl.BlockSpec((1, 1, D), lambda e, f: (e, 0, 0)),
        ],
        out_specs=pl.BlockSpec((T, D), lambda e, f: (0, 0)),
        out_shape=jax.ShapeDtypeStruct((T, D), jnp.float32),
    )(gate, h, W1, b1.reshape(E, 1, F), W2, b2.reshape(E, 1, D))

    return (out.reshape(B, S, D), logits.reshape(B, S, E),
            probs.reshape(B, S, E), zl.reshape(()), lb.reshape(()),
            load.reshape(E), counts.reshape(E))


# speedup vs baseline: 8.7921x; 1.3091x over previous
import functools

import jax
import jax.numpy as jnp
from jax import lax
from jax.experimental import pallas as pl
from jax.experimental.pallas import tpu as pltpu
from jax.experimental.pallas import tpu_sc as plsc

T = 2048
D = 768
F = 3072
E = 8
K = 2
BLK = 256
NBLK = (T * K) // BLK + E - 1
XR = NBLK * BLK
CB = 256
NCH = T // CB
NC = 2
NS = 16
NW = NC * NS
TPW = T // NW


def _router_kernel(h_ref, wg_ref, bg_ref,
                   logits_ref, probs_ref, zl_ref, lb_ref, load_ref,
                   counts_ref, pos1_ref, pos2_ref, w1b_ref, w2b_ref, bexp_ref,
                   cnt_ref, cum_ref):
    h = h_ref[...]
    logits = jnp.dot(h, wg_ref[...], preferred_element_type=jnp.float32)
    logits = logits + bg_ref[...]
    logits_ref[...] = logits

    m = jnp.max(logits, axis=1, keepdims=True)
    el = jnp.exp(logits - m)
    se = jnp.sum(el, axis=1, keepdims=True)
    probs = el / se
    probs_ref[...] = probs

    idx = lax.broadcasted_iota(jnp.int32, (T, E), 1)
    m1 = jnp.max(probs, axis=1, keepdims=True)
    a1 = jnp.min(jnp.where(probs == m1, idx, E), axis=1, keepdims=True)
    p2 = jnp.where(idx == a1, -jnp.inf, probs)
    m2 = jnp.max(p2, axis=1, keepdims=True)
    a2 = jnp.min(jnp.where(p2 == m2, idx, E), axis=1, keepdims=True)
    s = m1 + m2
    w1 = m1 / s
    w2 = m2 / s
    oh1 = (idx == a1).astype(jnp.float32)
    oh2 = (idx == a2).astype(jnp.float32)

    ones_l = jnp.ones((1, 128), jnp.float32)
    w1b_ref[...] = w1 * ones_l
    w2b_ref[...] = w2 * ones_l

    z = m + jnp.log(se)
    zl_ref[...] = jnp.sum(z * z, axis=0, keepdims=True) / T
    mean_probs = jnp.sum(probs, axis=0, keepdims=True) / T

    cnt_ref[...] = oh1 + oh2
    ri = lax.broadcasted_iota(jnp.int32, (CB, CB), 0)
    ci = lax.broadcasted_iota(jnp.int32, (CB, CB), 1)
    tri = (ci < ri).astype(jnp.float32)

    def body(i, carry):
        chunk = cnt_ref[pl.ds(i * CB, CB), :]
        csum = jnp.dot(tri, chunk, preferred_element_type=jnp.float32,
                       precision=lax.Precision.HIGHEST)
        cum_ref[pl.ds(i * CB, CB), :] = csum + carry
        return carry + jnp.sum(chunk, axis=0, keepdims=True)

    counts = lax.fori_loop(0, NCH, body, jnp.zeros((1, E), jnp.float32))
    counts_ref[...] = counts
    load = counts / (T * K)
    load_ref[...] = load
    lb_ref[...] = jnp.sum(load * mean_probs, axis=1, keepdims=True) * E

    cum = cum_ref[...]
    rank1 = jnp.sum(cum * oh1, axis=1, keepdims=True)
    rank2 = jnp.sum(cum * oh2, axis=1, keepdims=True)

    blkcnt = ((counts.astype(jnp.int32) + (BLK - 1)) // BLK).astype(jnp.float32)
    ui = lax.broadcasted_iota(jnp.int32, (E, E), 0)
    uj = lax.broadcasted_iota(jnp.int32, (E, E), 1)
    triu = (ui < uj).astype(jnp.float32)
    startblk = jnp.dot(blkcnt, triu, preferred_element_type=jnp.float32,
                       precision=lax.Precision.HIGHEST)
    start_rows = startblk * BLK
    pos1 = jnp.sum(start_rows * oh1, axis=1, keepdims=True) + rank1
    pos2 = jnp.sum(start_rows * oh2, axis=1, keepdims=True) + rank2
    pos1_ref[...] = pos1.astype(jnp.int32)
    pos2_ref[...] = pos2.astype(jnp.int32)

    endblk = startblk + blkcnt
    jj = lax.broadcasted_iota(jnp.int32, (NBLK, E), 0).astype(jnp.float32)
    mcnt = jnp.sum((endblk <= jj).astype(jnp.float32), axis=1, keepdims=True)
    bexp_ref[...] = jnp.minimum(mcnt, E - 1).astype(jnp.int32)


def _run_router(h, W_g, b_g):
    outs = pl.pallas_call(
        _router_kernel,
        out_shape=(
            jax.ShapeDtypeStruct((T, E), jnp.float32),
            jax.ShapeDtypeStruct((T, E), jnp.float32),
            jax.ShapeDtypeStruct((1, 1), jnp.float32),
            jax.ShapeDtypeStruct((1, 1), jnp.float32),
            jax.ShapeDtypeStruct((1, E), jnp.float32),
            jax.ShapeDtypeStruct((1, E), jnp.float32),
            jax.ShapeDtypeStruct((T, 1), jnp.int32),
            jax.ShapeDtypeStruct((T, 1), jnp.int32),
            jax.ShapeDtypeStruct((T, 128), jnp.float32),
            jax.ShapeDtypeStruct((T, 128), jnp.float32),
            jax.ShapeDtypeStruct((NBLK, 1), jnp.int32),
        ),
        scratch_shapes=[
            pltpu.VMEM((T, E), jnp.float32),
            pltpu.VMEM((T, E), jnp.float32),
        ],
    )(h, W_g, b_g.reshape(1, E))
    return outs


def _dispatch_body(h_hbm, p1_hbm, p2_hbm, w1b_hbm, w2b_hbm, x_hbm, w16_hbm,
                   idx1_v, idx2_v, rows_v, wb1_v, wb2_v, s1, s2, s3, s4):
    wid = lax.axis_index("s") * NC + lax.axis_index("c")
    base = wid * TPW
    pltpu.sync_copy(p1_hbm.at[pl.ds(base, TPW)], idx1_v)
    pltpu.sync_copy(p2_hbm.at[pl.ds(base, TPW)], idx2_v)
    pltpu.sync_copy(h_hbm.at[pl.ds(base, TPW)], rows_v)
    pltpu.sync_copy(w1b_hbm.at[pl.ds(base, TPW)], wb1_v)
    pltpu.sync_copy(w2b_hbm.at[pl.ds(base, TPW)], wb2_v)
    c1 = pltpu.async_copy(rows_v, x_hbm.at[idx1_v], s1)
    c2 = pltpu.async_copy(rows_v, x_hbm.at[idx2_v], s2)
    c3 = pltpu.async_copy(wb1_v, w16_hbm.at[idx1_v], s3)
    c4 = pltpu.async_copy(wb2_v, w16_hbm.at[idx2_v], s4)
    c1.wait()
    c2.wait()
    c3.wait()
    c4.wait()


def _sc_dispatch(h, pos1, pos2, w1b, w2b):
    mesh = plsc.VectorSubcoreMesh(core_axis_name="c", subcore_axis_name="s")
    fn = functools.partial(
        pl.kernel,
        mesh=mesh,
        out_type=(
            jax.ShapeDtypeStruct((XR, D), jnp.float32),
            jax.ShapeDtypeStruct((XR, 128), jnp.float32),
        ),
        scratch_types=[
            pltpu.VMEM((TPW,), jnp.int32),
            pltpu.VMEM((TPW,), jnp.int32),
            pltpu.VMEM((TPW, D), jnp.float32),
            pltpu.VMEM((TPW, 128), jnp.float32),
            pltpu.VMEM((TPW, 128), jnp.float32),
            pltpu.SemaphoreType.DMA,
            pltpu.SemaphoreType.DMA,
            pltpu.SemaphoreType.DMA,
            pltpu.SemaphoreType.DMA,
        ],
    )(_dispatch_body)
    return fn(h, pos1, pos2, w1b, w2b)


def _ffn_kernel(bexp_ref, x_ref, w16_ref, w1_ref, b1_ref, w2_ref, b2_ref,
                y_ref):
    mid = jnp.dot(x_ref[...].astype(jnp.bfloat16),
                  w1_ref[0].astype(jnp.bfloat16),
                  preferred_element_type=jnp.float32) + b1_ref[0]
    mid = jax.nn.gelu(mid)
    wcol = w16_ref[...][:, 0:1]
    mid = mid * wcol
    y = jnp.dot(mid.astype(jnp.bfloat16), w2_ref[0].astype(jnp.bfloat16),
                preferred_element_type=jnp.float32)
    y_ref[...] = y + wcol * b2_ref[0]


def _run_ffn(bexp, x, w16, W1, b1, W2, b2):
    grid_spec = pltpu.PrefetchScalarGridSpec(
        num_scalar_prefetch=1,
        grid=(NBLK,),
        in_specs=[
            pl.BlockSpec((BLK, D), lambda j, be: (j, 0)),
            pl.BlockSpec((BLK, 128), lambda j, be: (j, 0)),
            pl.BlockSpec((1, D, F), lambda j, be: (be[j], 0, 0)),
            pl.BlockSpec((1, 1, F), lambda j, be: (be[j], 0, 0)),
            pl.BlockSpec((1, F, D), lambda j, be: (be[j], 0, 0)),
            pl.BlockSpec((1, 1, D), lambda j, be: (be[j], 0, 0)),
        ],
        out_specs=pl.BlockSpec((BLK, D), lambda j, be: (j, 0)),
    )
    return pl.pallas_call(
        _ffn_kernel,
        grid_spec=grid_spec,
        out_shape=jax.ShapeDtypeStruct((XR, D), jnp.float32),
    )(bexp, x, w16, W1, b1.reshape(E, 1, F), W2, b2.reshape(E, 1, D))


def _combine_body(y_hbm, p1_hbm, p2_hbm, out_hbm,
                  idx1_v, idx2_v, rows1_v, rows2_v, s1, s2):
    wid = lax.axis_index("s") * NC + lax.axis_index("c")
    base = wid * TPW
    pltpu.sync_copy(p1_hbm.at[pl.ds(base, TPW)], idx1_v)
    pltpu.sync_copy(p2_hbm.at[pl.ds(base, TPW)], idx2_v)
    c1 = pltpu.async_copy(y_hbm.at[idx1_v], rows1_v, s1)
    c2 = pltpu.async_copy(y_hbm.at[idx2_v], rows2_v, s2)
    c1.wait()
    c2.wait()

    def tbody(t, _):
        def cbody(c, __):
            sl = pl.ds(c * 16, 16)
            rows1_v[t, sl] = rows1_v[t, sl] + rows2_v[t, sl]
            return __
        return lax.fori_loop(0, D // 16, cbody, _)

    lax.fori_loop(0, TPW, tbody, 0)
    pltpu.sync_copy(rows1_v, out_hbm.at[pl.ds(base, TPW)])


def _sc_combine(y, pos1, pos2):
    mesh = plsc.VectorSubcoreMesh(core_axis_name="c", subcore_axis_name="s")
    fn = functools.partial(
        pl.kernel,
        mesh=mesh,
        out_type=jax.ShapeDtypeStruct((T, D), jnp.float32),
        scratch_types=[
            pltpu.VMEM((TPW,), jnp.int32),
            pltpu.VMEM((TPW,), jnp.int32),
            pltpu.VMEM((TPW, D), jnp.float32),
            pltpu.VMEM((TPW, D), jnp.float32),
            pltpu.SemaphoreType.DMA,
            pltpu.SemaphoreType.DMA,
        ],
    )(_combine_body)
    return fn(y, pos1, pos2)


def kernel(h_t, W_g, b_g, W1, b1, W2, b2):
    B, S, _ = h_t.shape
    h = h_t.reshape(T, D)
    (logits, probs, zl, lb, load, counts,
     pos1, pos2, w1b, w2b, bexp) = _run_router(h, W_g, b_g)

    p1f = pos1.reshape(T)
    p2f = pos2.reshape(T)
    x, w16 = _sc_dispatch(h, p1f, p2f, w1b, w2b)
    y = _run_ffn(bexp.reshape(NBLK), x, w16, W1, b1, W2, b2)
    out = _sc_combine(y, p1f, p2f)

    return (out.reshape(B, S, D), logits.reshape(B, S, E),
            probs.reshape(B, S, E), zl.reshape(()), lb.reshape(()),
            load.reshape(E), counts.reshape(E))
